# Initial kernel scaffold; baseline (speedup 1.0000x reference)
import jax
import jax.numpy as jnp
from jax.experimental import pallas as pl


def kernel(inputs, support, edge_type, W1, W01, W2, W02):
    raise NotImplementedError("write your pallas kernel here")


# speedup vs baseline: 8.5960x; 8.5960x over previous
import functools

import jax
import jax.numpy as jnp
from jax import lax
from jax.experimental import pallas as pl
from jax.experimental.pallas import tpu as pltpu
from jax.experimental.pallas import tpu_sc as plsc

N = 10000
E = 320000
D = 128
R = 8

NC = 2
NS = 16
NW = NC * NS
B = 128
CH = 80
E_PAD = NW * CH * B
NACC = 10112
HALLOC = 80128

_mesh = plsc.VectorSubcoreMesh(core_axis_name="c", subcore_axis_name="s")
_f32 = jnp.float32
_i32 = jnp.int32


@functools.partial(
    pl.kernel,
    out_type=jax.ShapeDtypeStruct((NC, HALLOC, 16), _f32),
    mesh=_mesh,
    compiler_params=pltpu.CompilerParams(use_tc_tiling_on_sc=False),
    scratch_types=[
        pltpu.VMEM((CH, B), _i32),
        pltpu.VMEM((B, 16), _f32),
        pltpu.VMEM((1252, 16), _f32),
        pltpu.VMEM_SHARED((HALLOC, 16), _f32),
    ],
)
def _deg_kernel(key_hbm, out_hbm, key_v, ones_v, zbuf, hist):
    c = lax.axis_index("c")
    s = lax.axis_index("s")
    wid = s * NC + c

    def fill_row(i, _):
        ones_v[i, pl.ds(0, 16)] = jnp.ones((16,), _f32)
        return 0
    lax.fori_loop(0, B, fill_row, 0)

    def zero_row(i, _):
        zbuf[i, pl.ds(0, 16)] = jnp.zeros((16,), _f32)
        return 0
    lax.fori_loop(0, 1252, zero_row, 0)

    for t in range(4):
        pltpu.sync_copy(zbuf, hist.at[pl.ds(s * 5008 + t * 1252, 1252)])
    plsc.subcore_barrier()

    pltpu.sync_copy(key_hbm.at[wid], key_v)

    def chunk(j, _):
        pltpu.sync_copy(ones_v, hist.at[key_v.at[j]], add=True)
        return 0
    lax.fori_loop(0, CH, chunk, 0)
    plsc.subcore_barrier()

    pltpu.sync_copy(hist.at[pl.ds(s * 5008, 5008)],
                    out_hbm.at[c, pl.ds(s * 5008, 5008)])


@functools.partial(
    pl.kernel,
    out_type=jax.ShapeDtypeStruct((NC, NACC, D), _f32),
    mesh=_mesh,
    compiler_params=pltpu.CompilerParams(needs_layout_passes=False),
    scratch_types=[
        pltpu.VMEM((CH, B), _i32),
        pltpu.VMEM((CH, B), _i32),
        pltpu.VMEM((CH, B), _f32),
        pltpu.VMEM((B, D), _f32),
        pltpu.VMEM_SHARED((NACC, D), _f32),
        pltpu.SemaphoreType.DMA,
    ],
)
def _edge_kernel(xr_hbm, gidx_hbm, dst_hbm, w_hbm, out_hbm,
                 gidx_v, dst_v, w_v, rows_v, acc, sem):
    c = lax.axis_index("c")
    s = lax.axis_index("s")
    wid = s * NC + c

    def zero_row(i, _):
        for cb in range(8):
            rows_v[i, pl.ds(cb * 16, 16)] = jnp.zeros((16,), _f32)
        return 0
    lax.fori_loop(0, B, zero_row, 0)

    for t in range(4):
        pltpu.sync_copy(rows_v, acc.at[pl.ds(s * 632 + t * 128, 128)])
    pltpu.sync_copy(rows_v.at[pl.ds(0, 120)],
                    acc.at[pl.ds(s * 632 + 512, 120)])
    plsc.subcore_barrier()

    pltpu.sync_copy(gidx_hbm.at[wid], gidx_v)
    pltpu.sync_copy(dst_hbm.at[wid], dst_v)
    pltpu.sync_copy(w_hbm.at[wid], w_v)

    def chunk(j, _):
        pltpu.async_copy(xr_hbm.at[gidx_v.at[j]], rows_v, sem).wait()

        def scale_row(r, _):
            wv = plsc.load_gather(
                w_v, [jnp.full((16,), j, _i32), jnp.full((16,), r, _i32)])
            for cb in range(8):
                sl = pl.ds(cb * 16, 16)
                rows_v[r, sl] = rows_v[r, sl] * wv
            return 0
        lax.fori_loop(0, B, scale_row, 0)

        pltpu.sync_copy(rows_v, acc.at[dst_v.at[j]], add=True)
        return 0
    lax.fori_loop(0, CH, chunk, 0)
    plsc.subcore_barrier()

    pltpu.sync_copy(acc.at[pl.ds(s * 632, 632)],
                    out_hbm.at[c, pl.ds(s * 632, 632)])


def _xr_body(x_ref, w_ref, o_ref):
    o_ref[0] = jnp.dot(x_ref[...], w_ref[0], preferred_element_type=_f32)


def _per_relation_transform(x, wr):
    return pl.pallas_call(
        _xr_body,
        grid=(R, 5),
        in_specs=[
            pl.BlockSpec((2000, D), lambda r, i: (i, 0)),
            pl.BlockSpec((1, D, D), lambda r, i: (r, 0, 0)),
        ],
        out_specs=pl.BlockSpec((1, 2000, D), lambda r, i: (r, i, 0)),
        out_shape=jax.ShapeDtypeStruct((R, N, D), _f32),
    )(x, wr)


def _combine_body(x_ref, w_ref, a_ref, o_ref, *, relu):
    y = jnp.dot(x_ref[...], w_ref[...], preferred_element_type=_f32)
    y = y + a_ref[0] + a_ref[1]
    o_ref[...] = jnp.maximum(y, 0.0) if relu else y


def _combine(x, w0, acc2, relu):
    return pl.pallas_call(
        functools.partial(_combine_body, relu=relu),
        grid=(5,),
        in_specs=[
            pl.BlockSpec((2000, D), lambda i: (i, 0)),
            pl.BlockSpec((D, D), lambda i: (0, 0)),
            pl.BlockSpec((NC, 2000, D), lambda i: (0, i, 0)),
        ],
        out_specs=pl.BlockSpec((2000, D), lambda i: (i, 0)),
        out_shape=jax.ShapeDtypeStruct((N, D), _f32),
    )(x, w0, acc2)


def _pad_tiles(a, fill):
    pad = jnp.full((E_PAD - E,), fill, a.dtype)
    return jnp.concatenate([a, pad]).reshape(NW, CH, B)


def kernel(inputs, support, edge_type, W1, W01, W2, W02):
    src = support[0]
    dst = support[1]
    key = dst * R + edge_type

    hist2 = _deg_kernel(_pad_tiles(key, N * R))
    deg = hist2[0, :, 0] + hist2[1, :, 0]
    inv = 1.0 / jnp.maximum(deg, 1.0)
    w = inv[key]

    gidx_pad = _pad_tiles(edge_type * N + src, 0)
    dst_pad = _pad_tiles(dst, N)
    w_pad = _pad_tiles(w, 0.0)

    def layer(x, wr, w0, relu):
        xr = _per_relation_transform(x, wr).reshape(R * N, D)
        acc2 = _edge_kernel(xr, gidx_pad, dst_pad, w_pad)
        return _combine(x, w0, acc2, relu)

    h = layer(inputs, W1, W01, True)
    return layer(h, W2, W02, False)
